# initial kernel scaffold (unmeasured)
import jax
import jax.numpy as jnp
from jax import lax
from jax.experimental import pallas as pl
from jax.experimental.pallas import tpu as pltpu

N_DEV = 8
N_TOK = 1024
D_IN = 512
D_OUT = 1024
N_EXP = 32
E_LOC = N_EXP // N_DEV
CHUNK = N_TOK // N_DEV
N_STEP = N_DEV - 1


def kernel(x, router_W, route_idx, expert_W):
    def body(x_ref, rw_ref, idx_ref, ew_ref, out_ref,
             comm_ref, rs_send_sems, rs_recv_sems, ag_send_sems, ag_recv_sems):
        my = lax.axis_index("i")
        right = lax.rem(my + 1, N_DEV)

        xf = x_ref[...]
        scores = jnp.dot(xf, rw_ref[...], preferred_element_type=jnp.float32)
        m = jnp.max(scores, axis=1, keepdims=True)
        p = jnp.exp(scores - m)
        e_ids = lax.broadcasted_iota(jnp.int32, (N_TOK, N_EXP), 1)
        idx = idx_ref[...]
        sel = (e_ids == idx[:, 0:1]) | (e_ids == idx[:, 1:2])
        psel = jnp.where(sel, p, 0.0)
        gs = jnp.sum(psel, axis=1, keepdims=True)

        acc = jnp.zeros((N_TOK, D_OUT), jnp.float32)
        for e in range(E_LOC):
            ge = my * E_LOC + e
            w = jnp.sum(jnp.where(e_ids == ge, psel, 0.0), axis=1,
                        keepdims=True) / gs
            xw = (xf * w).astype(jnp.bfloat16)
            we = ew_ref[e].astype(jnp.bfloat16)
            acc = acc + jnp.dot(xw, we, preferred_element_type=jnp.float32)
        out_ref[...] = acc

        for s in range(N_STEP):
            c_send = lax.rem(my - s + N_DEV, N_DEV)
            c_recv = lax.rem(my - s - 1 + N_DEV, N_DEV)
            rdma = pltpu.make_async_remote_copy(
                src_ref=out_ref.at[pl.ds(c_send * CHUNK, CHUNK), :],
                dst_ref=comm_ref.at[s],
                send_sem=rs_send_sems.at[s],
                recv_sem=rs_recv_sems.at[s],
                device_id=(right,),
                device_id_type=pl.DeviceIdType.MESH,
            )
            rdma.start()
            rdma.wait()
            out_ref[pl.ds(c_recv * CHUNK, CHUNK), :] = (
                out_ref[pl.ds(c_recv * CHUNK, CHUNK), :] + comm_ref[s]
            )

        for t in range(N_STEP):
            c_send = lax.rem(my + 1 - t + N_DEV, N_DEV)
            rdma = pltpu.make_async_remote_copy(
                src_ref=out_ref.at[pl.ds(c_send * CHUNK, CHUNK), :],
                dst_ref=out_ref.at[pl.ds(c_send * CHUNK, CHUNK), :],
                send_sem=ag_send_sems.at[t],
                recv_sem=ag_recv_sems.at[t],
                device_id=(right,),
                device_id_type=pl.DeviceIdType.MESH,
            )
            rdma.start()
            rdma.wait()

    return pl.pallas_call(
        body,
        out_shape=jax.ShapeDtypeStruct((N_TOK, D_OUT), jnp.float32),
        in_specs=[
            pl.BlockSpec(memory_space=pltpu.VMEM),
            pl.BlockSpec(memory_space=pltpu.VMEM),
            pl.BlockSpec(memory_space=pltpu.VMEM),
            pl.BlockSpec(memory_space=pltpu.VMEM),
        ],
        out_specs=pl.BlockSpec(memory_space=pltpu.VMEM),
        scratch_shapes=[
            pltpu.VMEM((N_STEP, CHUNK, D_OUT), jnp.float32),
            pltpu.SemaphoreType.DMA((N_STEP,)),
            pltpu.SemaphoreType.DMA((N_STEP,)),
            pltpu.SemaphoreType.DMA((N_STEP,)),
            pltpu.SemaphoreType.DMA((N_STEP,)),
        ],
        compiler_params=pltpu.CompilerParams(collective_id=0),
    )(x, router_W, route_idx, expert_W)


# baseline (device time: 127077 ns/iter reference)
import jax
import jax.numpy as jnp
from jax import lax
from jax.experimental import pallas as pl
from jax.experimental.pallas import tpu as pltpu

N_DEV = 8
N_TOK = 1024
D_IN = 512
D_OUT = 1024
N_EXP = 32
E_LOC = N_EXP // N_DEV
CHUNK = N_TOK // N_DEV
N_STEP = N_DEV - 1


def kernel(x, router_W, route_idx, expert_W):
    def body(x_ref, rw_ref, idx_ref, ew_ref, out_ref,
             comm_ref, rs_send_sems, rs_recv_sems, ag_send_sems, ag_recv_sems):
        my = lax.axis_index("i")
        right = lax.rem(my + 1, N_DEV)

        xf = x_ref[...]
        scores = jnp.dot(xf, rw_ref[...], preferred_element_type=jnp.float32)
        m = jnp.max(scores, axis=1, keepdims=True)
        p = jnp.exp(scores - m)
        e_ids = lax.broadcasted_iota(jnp.int32, (N_TOK, N_EXP), 1)
        idx = idx_ref[...]
        sel = (e_ids == idx[:, 0:1]) | (e_ids == idx[:, 1:2])
        psel = jnp.where(sel, p, 0.0)
        gs = jnp.sum(psel, axis=1, keepdims=True)

        acc = jnp.zeros((N_TOK, D_OUT), jnp.float32)
        for e in range(E_LOC):
            ge = my * E_LOC + e
            w = jnp.sum(jnp.where(e_ids == ge, psel, 0.0), axis=1,
                        keepdims=True) / gs
            xw = (xf * w).astype(jnp.bfloat16)
            we = ew_ref[e].astype(jnp.bfloat16)
            acc = acc + jnp.dot(xw, we, preferred_element_type=jnp.float32)
        out_ref[...] = acc

        for s in range(N_STEP):
            c_send = lax.rem(my - s + N_DEV, N_DEV)
            c_recv = lax.rem(my - s - 1 + N_DEV, N_DEV)
            rdma = pltpu.make_async_remote_copy(
                src_ref=out_ref.at[pl.ds(c_send * CHUNK, CHUNK), :],
                dst_ref=comm_ref.at[s],
                send_sem=rs_send_sems.at[s],
                recv_sem=rs_recv_sems.at[s],
                device_id=(right,),
                device_id_type=pl.DeviceIdType.MESH,
            )
            rdma.start()
            rdma.wait()
            out_ref[pl.ds(c_recv * CHUNK, CHUNK), :] = (
                out_ref[pl.ds(c_recv * CHUNK, CHUNK), :] + comm_ref[s]
            )

        for t in range(N_STEP):
            c_send = lax.rem(my + 1 - t + N_DEV, N_DEV)
            rdma = pltpu.make_async_remote_copy(
                src_ref=out_ref.at[pl.ds(c_send * CHUNK, CHUNK), :],
                dst_ref=out_ref.at[pl.ds(c_send * CHUNK, CHUNK), :],
                send_sem=ag_send_sems.at[t],
                recv_sem=ag_recv_sems.at[t],
                device_id=(right,),
                device_id_type=pl.DeviceIdType.MESH,
            )
            rdma.start()
            rdma.wait()

    return pl.pallas_call(
        body,
        out_shape=jax.ShapeDtypeStruct((N_TOK, D_OUT), jnp.float32),
        in_specs=[
            pl.BlockSpec(memory_space=pltpu.VMEM),
            pl.BlockSpec(memory_space=pltpu.VMEM),
            pl.BlockSpec(memory_space=pltpu.VMEM),
            pl.BlockSpec(memory_space=pltpu.VMEM),
        ],
        out_specs=pl.BlockSpec(memory_space=pltpu.VMEM),
        scratch_shapes=[
            pltpu.VMEM((N_STEP, CHUNK, D_OUT), jnp.float32),
            pltpu.SemaphoreType.DMA((N_STEP,)),
            pltpu.SemaphoreType.DMA((N_STEP,)),
            pltpu.SemaphoreType.DMA((N_STEP,)),
            pltpu.SemaphoreType.DMA((N_STEP,)),
        ],
    )(x, router_W, route_idx, expert_W)
